# baseline (speedup 1.0000x reference)
import functools

import jax
import jax.numpy as jnp
from jax import lax
from jax.experimental import pallas as pl
from jax.experimental.pallas import tpu as pltpu
from jax.experimental.pallas import tpu_sc as plsc

B = 16384
D = 128
NC = 2
NS = 16
NW = NC * NS
BPW = B // NW
C = 64
NCH = BPW // C
LANES = 16


def _body(heads_hbm, rels_hbm, tails_hbm, ent_hbm, rel_hbm, out_hbm,
          hidx, ridx, tidx, hb, rb, tb, ob, rel_sh,
          sem_g0, sem_g1, sem_o0, sem_o1, sem_r0, sem_r1):
    sid = lax.axis_index("s")
    wid = sid * NC + lax.axis_index("c")
    base = wid * BPW

    @pl.when(sid == 0)
    def _():
        pltpu.sync_copy(rel_hbm, rel_sh)

    pltpu.sync_copy(heads_hbm.at[pl.ds(base, BPW)], hidx)
    pltpu.sync_copy(rels_hbm.at[pl.ds(base, BPW)], ridx)
    pltpu.sync_copy(tails_hbm.at[pl.ds(base, BPW)], tidx)
    plsc.subcore_barrier()

    sg = (sem_g0, sem_g1)
    so = (sem_o0, sem_o1)
    sr = (sem_r0, sem_r1)

    def start_gather(g, p):
        off = g * C
        return (
            pltpu.async_copy(ent_hbm.at[hidx.at[pl.ds(off, C)]], hb.at[p], sg[p]),
            pltpu.async_copy(rel_sh.at[ridx.at[pl.ds(off, C)]], rb.at[p], sr[p]),
            pltpu.async_copy(ent_hbm.at[tidx.at[pl.ds(off, C)]], tb.at[p], sg[p]),
        )

    inflight = [None, None]
    outflight = [None, None]
    inflight[0] = start_gather(0, 0)

    for g in range(NCH):
        p = g & 1
        if g + 1 < NCH:
            inflight[1 - p] = start_gather(g + 1, 1 - p)
        for cp in inflight[p]:
            cp.wait()
        if outflight[p] is not None:
            outflight[p].wait()

        hp, rp, tp, op = hb.at[p], rb.at[p], tb.at[p], ob.at[p]

        @plsc.parallel_loop(0, C, unroll=4)
        def _row(r):
            for d in range(D // LANES):
                sl = pl.ds(d * LANES, LANES)
                op[r, sl] = hp[r, sl] + rp[r, sl] - tp[r, sl]
        outflight[p] = pltpu.async_copy(
            op, out_hbm.at[pl.ds(base + g * C, C)], so[p])

    for p in range(2):
        if outflight[p] is not None:
            outflight[p].wait()


def kernel(heads, relations, tails, entity_table, relation_table):
    mesh = plsc.VectorSubcoreMesh(core_axis_name="c", subcore_axis_name="s")
    k = functools.partial(
        pl.kernel,
        mesh=mesh,
        out_type=jax.ShapeDtypeStruct((B, D), jnp.float32),
        scratch_types=[
            pltpu.VMEM((BPW,), jnp.int32),
            pltpu.VMEM((BPW,), jnp.int32),
            pltpu.VMEM((BPW,), jnp.int32),
            pltpu.VMEM((2, C, D), jnp.float32),
            pltpu.VMEM((2, C, D), jnp.float32),
            pltpu.VMEM((2, C, D), jnp.float32),
            pltpu.VMEM((2, C, D), jnp.float32),
            pltpu.VMEM_SHARED((1000, D), jnp.float32),
            pltpu.SemaphoreType.DMA,
            pltpu.SemaphoreType.DMA,
            pltpu.SemaphoreType.DMA,
            pltpu.SemaphoreType.DMA,
            pltpu.SemaphoreType.DMA,
            pltpu.SemaphoreType.DMA,
        ],
    )(_body)
    return k(heads.astype(jnp.int32), relations.astype(jnp.int32),
             tails.astype(jnp.int32), entity_table, relation_table)


# speedup vs baseline: 1.1506x; 1.1506x over previous
import functools

import jax
import jax.numpy as jnp
from jax import lax
from jax.experimental import pallas as pl
from jax.experimental.pallas import tpu as pltpu
from jax.experimental.pallas import tpu_sc as plsc

B = 16384
D = 128
NC = 2
NS = 16
NW = NC * NS
BPW = B // NW
C = 64
NCH = BPW // C
NB = 3
LANES = 16


def _body(heads_hbm, rels_hbm, tails_hbm, ent_hbm, rel_hbm, out_hbm,
          hidx, ridx, tidx, hb, rb, tb, ob, rel_sh,
          sem_g0, sem_g1, sem_g2, sem_o0, sem_o1,
          sem_r0, sem_r1, sem_r2):
    sid = lax.axis_index("s")
    wid = sid * NC + lax.axis_index("c")
    base = wid * BPW

    @pl.when(sid == 0)
    def _():
        pltpu.sync_copy(rel_hbm, rel_sh)

    pltpu.sync_copy(heads_hbm.at[pl.ds(base, BPW)], hidx)
    pltpu.sync_copy(rels_hbm.at[pl.ds(base, BPW)], ridx)
    pltpu.sync_copy(tails_hbm.at[pl.ds(base, BPW)], tidx)
    plsc.subcore_barrier()

    sg = (sem_g0, sem_g1, sem_g2)
    sr = (sem_r0, sem_r1, sem_r2)
    so = (sem_o0, sem_o1)

    def start_gather(g, p):
        off = g * C
        return (
            pltpu.async_copy(ent_hbm.at[hidx.at[pl.ds(off, C)]], hb.at[p], sg[p]),
            pltpu.async_copy(rel_sh.at[ridx.at[pl.ds(off, C)]], rb.at[p], sr[p]),
            pltpu.async_copy(ent_hbm.at[tidx.at[pl.ds(off, C)]], tb.at[p], sg[p]),
        )

    inflight = [None] * NB
    outflight = [None, None]
    for g in range(min(NB - 1, NCH)):
        inflight[g % NB] = start_gather(g, g % NB)

    for g in range(NCH):
        p = g % NB
        q = g & 1
        if g + NB - 1 < NCH:
            inflight[(g + NB - 1) % NB] = start_gather(g + NB - 1, (g + NB - 1) % NB)
        for cp in inflight[p]:
            cp.wait()
        if outflight[q] is not None:
            outflight[q].wait()

        hp, rp, tp, op = hb.at[p], rb.at[p], tb.at[p], ob.at[q]

        def row(r, rc):
            for d in range(D // LANES):
                sl = pl.ds(d * LANES, LANES)
                op[r, sl] = hp[r, sl] + rp[r, sl] - tp[r, sl]
            return rc

        lax.fori_loop(0, C, row, 0)
        outflight[q] = pltpu.async_copy(
            op, out_hbm.at[pl.ds(base + g * C, C)], so[q])

    for q in range(2):
        if outflight[q] is not None:
            outflight[q].wait()


def kernel(heads, relations, tails, entity_table, relation_table):
    mesh = plsc.VectorSubcoreMesh(core_axis_name="c", subcore_axis_name="s")
    k = functools.partial(
        pl.kernel,
        mesh=mesh,
        out_type=jax.ShapeDtypeStruct((B, D), jnp.float32),
        scratch_types=[
            pltpu.VMEM((BPW,), jnp.int32),
            pltpu.VMEM((BPW,), jnp.int32),
            pltpu.VMEM((BPW,), jnp.int32),
            pltpu.VMEM((NB, C, D), jnp.float32),
            pltpu.VMEM((NB, C, D), jnp.float32),
            pltpu.VMEM((NB, C, D), jnp.float32),
            pltpu.VMEM((2, C, D), jnp.float32),
            pltpu.VMEM_SHARED((1000, D), jnp.float32),
            pltpu.SemaphoreType.DMA,
            pltpu.SemaphoreType.DMA,
            pltpu.SemaphoreType.DMA,
            pltpu.SemaphoreType.DMA,
            pltpu.SemaphoreType.DMA,
            pltpu.SemaphoreType.DMA,
            pltpu.SemaphoreType.DMA,
            pltpu.SemaphoreType.DMA,
        ],
    )(_body)
    return k(heads.astype(jnp.int32), relations.astype(jnp.int32),
             tails.astype(jnp.int32), entity_table, relation_table)


# speedup vs baseline: 1.1790x; 1.0247x over previous
import functools

import jax
import jax.numpy as jnp
from jax import lax
from jax.experimental import pallas as pl
from jax.experimental.pallas import tpu as pltpu
from jax.experimental.pallas import tpu_sc as plsc

B = 16384
D = 128
NC = 2
NS = 16
NW = NC * NS
BPW = B // NW
C = 64
NCH = BPW // C
NB = 4
LANES = 16


def _body(heads_hbm, rels_hbm, tails_hbm, ent_hbm, rel_hbm, out_hbm,
          hidx, ridx, tidx, ob, rb, tb, rel_sh,
          sem_g0, sem_g1, sem_g2, sem_g3,
          sem_r0, sem_r1, sem_r2, sem_r3,
          sem_o0, sem_o1, sem_o2, sem_o3):
    sid = lax.axis_index("s")
    wid = sid * NC + lax.axis_index("c")
    base = wid * BPW

    @pl.when(sid == 0)
    def _():
        pltpu.sync_copy(rel_hbm, rel_sh)

    pltpu.sync_copy(heads_hbm.at[pl.ds(base, BPW)], hidx)
    pltpu.sync_copy(rels_hbm.at[pl.ds(base, BPW)], ridx)
    pltpu.sync_copy(tails_hbm.at[pl.ds(base, BPW)], tidx)
    plsc.subcore_barrier()

    sg = (sem_g0, sem_g1, sem_g2, sem_g3)
    sr = (sem_r0, sem_r1, sem_r2, sem_r3)
    so = (sem_o0, sem_o1, sem_o2, sem_o3)

    def start_gather(g):
        p = g % NB
        off = g * C
        return (
            pltpu.async_copy(ent_hbm.at[hidx.at[pl.ds(off, C)]], ob.at[p], sg[p]),
            pltpu.async_copy(rel_sh.at[ridx.at[pl.ds(off, C)]], rb.at[p], sr[p]),
            pltpu.async_copy(ent_hbm.at[tidx.at[pl.ds(off, C)]], tb.at[p], sg[p]),
        )

    inflight = [None] * NB
    outflight = [None] * NB
    for g in range(min(NB - 1, NCH)):
        inflight[g % NB] = start_gather(g)

    for g in range(NCH):
        p = g % NB
        for cp in inflight[p]:
            cp.wait()

        rp, tp, op = rb.at[p], tb.at[p], ob.at[p]

        def row(r, rc):
            for d in range(D // LANES):
                sl = pl.ds(d * LANES, LANES)
                plsc.addupdate(op.at[r, sl], rp[r, sl] - tp[r, sl])
            return rc

        lax.fori_loop(0, C, row, 0)
        outflight[p] = pltpu.async_copy(
            op, out_hbm.at[pl.ds(base + g * C, C)], so[p])

        gn = g + NB - 1
        if gn < NCH:
            pn = gn % NB
            if outflight[pn] is not None:
                outflight[pn].wait()
            inflight[pn] = start_gather(gn)

    for p in range(NB):
        if outflight[p] is not None:
            outflight[p].wait()


def kernel(heads, relations, tails, entity_table, relation_table):
    mesh = plsc.VectorSubcoreMesh(core_axis_name="c", subcore_axis_name="s")
    k = functools.partial(
        pl.kernel,
        mesh=mesh,
        out_type=jax.ShapeDtypeStruct((B, D), jnp.float32),
        scratch_types=[
            pltpu.VMEM((BPW,), jnp.int32),
            pltpu.VMEM((BPW,), jnp.int32),
            pltpu.VMEM((BPW,), jnp.int32),
            pltpu.VMEM((NB, C, D), jnp.float32),
            pltpu.VMEM((NB, C, D), jnp.float32),
            pltpu.VMEM((NB, C, D), jnp.float32),
            pltpu.VMEM_SHARED((1000, D), jnp.float32),
            pltpu.SemaphoreType.DMA,
            pltpu.SemaphoreType.DMA,
            pltpu.SemaphoreType.DMA,
            pltpu.SemaphoreType.DMA,
            pltpu.SemaphoreType.DMA,
            pltpu.SemaphoreType.DMA,
            pltpu.SemaphoreType.DMA,
            pltpu.SemaphoreType.DMA,
            pltpu.SemaphoreType.DMA,
            pltpu.SemaphoreType.DMA,
            pltpu.SemaphoreType.DMA,
            pltpu.SemaphoreType.DMA,
        ],
    )(_body)
    return k(heads.astype(jnp.int32), relations.astype(jnp.int32),
             tails.astype(jnp.int32), entity_table, relation_table)
